# baseline (speedup 1.0000x reference)
import functools

import jax
import jax.numpy as jnp
from jax import lax
from jax.experimental import pallas as pl
from jax.experimental.pallas import tpu as pltpu
import jax.experimental.pallas.tpu_sc as plsc


B = 128
N = 2000
NP = 2048
C = 32
D = 128
NW = 32
ROWS_PER_WORKER = B // NW
CHUNK = 128
NCHUNK = NP // CHUNK


def _prep_body(geo_ref, phi_ref, rsrp_ref, emb_ref, out_ref):
    mx = geo_ref[:, 0:1]
    my = geo_ref[:, 1:2]
    c00 = geo_ref[:, 2:3]
    c11 = geo_ref[:, 3:4]
    cs = geo_ref[:, 4:5]
    opa = geo_ref[:, 5:6]
    cidx = lax.broadcasted_iota(jnp.int32, (NP, C), 1)
    cx = (cidx % 4).astype(jnp.float32)
    cy = (cidx // 4).astype(jnp.float32)
    dx0 = cx - mx
    dx1 = cy - my
    q = dx0 * dx0 * c00 + dx1 * dx1 * c11 + dx0 * dx1 * cs
    gw = jnp.exp(-0.5 * q * 0.001)
    m = jnp.minimum(opa * emb_ref[:] * gw * gw, 0.99)
    phi = phi_ref[:]
    out_ref[:, 0:C] = m * jnp.cos(phi)
    out_ref[:, C:2 * C] = m * jnp.sin(phi)
    out_ref[:, 2 * C:3 * C] = rsrp_ref[:]
    out_ref[:, 3 * C:4 * C] = jnp.zeros((NP, C), jnp.float32)


def _prep(geo, phi_p, rsrp_p, emb_p):
    return pl.pallas_call(
        _prep_body,
        out_shape=jax.ShapeDtypeStruct((NP, D), jnp.float32),
    )(geo, phi_p, rsrp_p, emb_p)


NROUNDS = sum(range(1, 12))


def _sort_body(keys_ref, idx_ref):
    lane1 = lax.broadcasted_iota(jnp.int32, (1, NP), 1)
    keys0 = keys_ref[:]
    idx0 = lax.broadcasted_iota(jnp.int32, (B, NP), 1)

    def round_body(r, carry):
        k, j, both = carry
        s = lax.shift_left(1, j)
        low = (lane1 & s) == 0
        asc = (lane1 & lax.shift_left(1, k)) == 0
        want_self = asc == low
        partner = jnp.where(low, pltpu.roll(both, NP - s, 1),
                            pltpu.roll(both, s, 1))
        keys, idxf = both[:B], both[B:]
        pk, pif = partner[:B], partner[B:]
        idx = lax.bitcast_convert_type(idxf, jnp.int32)
        pi = lax.bitcast_convert_type(pif, jnp.int32)
        a_lt = (keys < pk) | ((keys == pk) & (idx < pi))
        take_self = a_lt == want_self
        both = jnp.where(jnp.concatenate([take_self, take_self], axis=0),
                         both, partner)
        wrap = j == 0
        k2 = jnp.where(wrap, k + 1, k)
        j2 = jnp.where(wrap, k, j - 1)
        return k2, j2, both

    both0 = jnp.concatenate(
        [keys0, lax.bitcast_convert_type(idx0, jnp.float32)], axis=0)
    _, _, both = lax.fori_loop(
        0, NROUNDS, round_body, (jnp.int32(1), jnp.int32(0), both0))
    idx_ref[:] = lax.bitcast_convert_type(both[B:], jnp.int32)


def _sort(keys):
    return pl.pallas_call(
        _sort_body,
        out_shape=jax.ShapeDtypeStruct((B, NP), jnp.int32),
    )(keys)


def _sc_body(table_hbm, idx_hbm, out_hbm, idx_row, buf0, buf1, out_v, sem0, sem1):
    wid = lax.axis_index("s") * 2 + lax.axis_index("c")
    bufs = (buf0, buf1)
    sems = (sem0, sem1)

    def start(slot, ch):
        pltpu.async_copy(
            table_hbm.at[idx_row.at[pl.ds(ch * CHUNK, CHUNK)]],
            bufs[slot], sems[slot])

    def wait(slot, ch):
        pltpu.make_async_copy(
            table_hbm.at[idx_row.at[pl.ds(ch * CHUNK, CHUNK)]],
            bufs[slot], sems[slot]).wait()

    def scan_chunk(buf, carry):
        def step(r, cy):
            (Trl, Trh, Til, Tih,
             Arl, Arh, Ail, Aih,
             Rrl, Rrh, Ril, Rih) = cy
            arl = buf[r, pl.ds(0, 16)]
            arh = buf[r, pl.ds(16, 16)]
            ail = buf[r, pl.ds(32, 16)]
            aih = buf[r, pl.ds(48, 16)]
            rrl = buf[r, pl.ds(64, 16)]
            rrh = buf[r, pl.ds(80, 16)]
            srl = arl * Trl - ail * Til
            sil = arl * Til + ail * Trl
            srh = arh * Trh - aih * Tih
            sih = arh * Tih + aih * Trh
            Arl = Arl + srl
            Ail = Ail + sil
            Arh = Arh + srh
            Aih = Aih + sih
            Rrl = Rrl + srl * rrl
            Ril = Ril + sil * rrl
            Rrh = Rrh + srh * rrh
            Rih = Rih + sih * rrh
            nTrl = Trl - srl
            nTil = Til - sil
            nTrh = Trh - srh
            nTih = Tih - sih
            return (nTrl, nTrh, nTil, nTih,
                    Arl, Arh, Ail, Aih,
                    Rrl, Rrh, Ril, Rih)

        return lax.fori_loop(0, CHUNK, step, carry)

    for rowi in range(ROWS_PER_WORKER):
        b = wid * ROWS_PER_WORKER + rowi
        pltpu.sync_copy(idx_hbm.at[b], idx_row)
        start(0, 0)

        ones = jnp.ones((16,), jnp.float32)
        zeros = jnp.zeros((16,), jnp.float32)
        carry = (ones, ones, zeros, zeros,
                 zeros, zeros, zeros, zeros,
                 zeros, zeros, zeros, zeros)

        def outer(ch2, cy):
            ch0 = ch2 * 2
            wait(0, ch0)
            start(1, ch0 + 1)
            cy = scan_chunk(buf0, cy)
            wait(1, ch0 + 1)

            @pl.when(ch2 < NCHUNK // 2 - 1)
            def _():
                start(0, ch0 + 2)

            return scan_chunk(buf1, cy)

        carry = lax.fori_loop(0, NCHUNK // 2, outer, carry)
        (_, _, _, _, Arl, Arh, Ail, Aih, Rrl, Rrh, Ril, Rih) = carry

        rel = Rrl + 1.0 - Arl
        reh = Rrh + 1.0 - Arh
        iml = Ril - Ail
        imh = Rih - Aih
        out_v[pl.ds(0, 16)] = rel * rel + iml * iml
        out_v[pl.ds(16, 16)] = reh * reh + imh * imh
        pltpu.sync_copy(out_v, out_hbm.at[b, :])


def _sc_call(table, sidx):
    fn = pl.kernel(
        _sc_body,
        out_type=jax.ShapeDtypeStruct((B, C), jnp.float32),
        mesh=plsc.VectorSubcoreMesh(
            core_axis_name="c", subcore_axis_name="s",
            num_cores=2, num_subcores=16,
        ),
        scratch_types=[
            pltpu.VMEM((NP,), jnp.int32),
            pltpu.VMEM((CHUNK, D), jnp.float32),
            pltpu.VMEM((CHUNK, D), jnp.float32),
            pltpu.VMEM((C,), jnp.float32),
            pltpu.SemaphoreType.DMA,
            pltpu.SemaphoreType.DMA,
        ],
    )
    return fn(table, sidx)


def kernel(position_grid, means2D, cov2d, rsrp, opacity, phi_o, depths, embedded):
    del position_grid
    padn = NP - N
    mx = jnp.pad(means2D[:, 0], (0, padn))
    my = jnp.pad(means2D[:, 1], (0, padn))
    c00 = jnp.pad(cov2d[:, 0, 0], (0, padn))
    c11 = jnp.pad(cov2d[:, 1, 1], (0, padn))
    cs = jnp.pad(cov2d[:, 0, 1] + cov2d[:, 1, 0], (0, padn))
    opa = jnp.pad(opacity[:, 0], (0, padn))
    zcol = jnp.zeros((NP,), jnp.float32)
    geo = jnp.stack([mx, my, c00, c11, cs, opa, zcol, zcol], axis=1)
    phi_p = jnp.pad(phi_o, ((0, padn), (0, 0)))
    rsrp_p = jnp.pad(rsrp, ((0, padn), (0, 0)))
    emb_p = jnp.pad(embedded, ((0, padn), (0, 0)))

    table = _prep(geo, phi_p, rsrp_p, emb_p)
    keys = jnp.pad(depths[..., 0], ((0, 0), (0, padn)),
                   constant_values=1e30)
    sidx = _sort(keys)
    return _sc_call(table, sidx)


# speedup vs baseline: 1.0084x; 1.0084x over previous
import functools

import jax
import jax.numpy as jnp
from jax import lax
from jax.experimental import pallas as pl
from jax.experimental.pallas import tpu as pltpu
import jax.experimental.pallas.tpu_sc as plsc


B = 128
N = 2000
NP = 2048
C = 32
D = 128
NW = 32
ROWS_PER_WORKER = B // NW
CHUNK = 128
NCHUNK = NP // CHUNK


def _prep_body(geo_ref, phi_ref, rsrp_ref, emb_ref, out_ref):
    mx = geo_ref[:, 0:1]
    my = geo_ref[:, 1:2]
    c00 = geo_ref[:, 2:3]
    c11 = geo_ref[:, 3:4]
    cs = geo_ref[:, 4:5]
    opa = geo_ref[:, 5:6]
    cidx = lax.broadcasted_iota(jnp.int32, (NP, C), 1)
    cx = (cidx % 4).astype(jnp.float32)
    cy = (cidx // 4).astype(jnp.float32)
    dx0 = cx - mx
    dx1 = cy - my
    q = dx0 * dx0 * c00 + dx1 * dx1 * c11 + dx0 * dx1 * cs
    gw = jnp.exp(-0.5 * q * 0.001)
    m = jnp.minimum(opa * emb_ref[:] * gw * gw, 0.99)
    phi = phi_ref[:]
    out_ref[:, 0:C] = m * jnp.cos(phi)
    out_ref[:, C:2 * C] = m * jnp.sin(phi)
    out_ref[:, 2 * C:3 * C] = rsrp_ref[:]
    out_ref[:, 3 * C:4 * C] = jnp.zeros((NP, C), jnp.float32)


def _prep(geo, phi_p, rsrp_p, emb_p):
    return pl.pallas_call(
        _prep_body,
        out_shape=jax.ShapeDtypeStruct((NP, D), jnp.float32),
    )(geo, phi_p, rsrp_p, emb_p)


NROUNDS = sum(range(1, 12))


def _sort_body(keys_ref, idx_ref):
    lane1 = lax.broadcasted_iota(jnp.int32, (1, NP), 1)
    keys0 = keys_ref[:]
    idx0 = lax.broadcasted_iota(jnp.int32, (B, NP), 1)

    def round_body(r, carry):
        k, j, keys, idx = carry
        s = lax.shift_left(1, j)
        low = (lane1 & s) == 0
        asc = (lane1 & lax.shift_left(1, k)) == 0
        want_self = asc == low
        pk = jnp.where(low, pltpu.roll(keys, NP - s, 1), pltpu.roll(keys, s, 1))
        pi = jnp.where(low, pltpu.roll(idx, NP - s, 1), pltpu.roll(idx, s, 1))
        a_lt = (keys < pk) | ((keys == pk) & (idx < pi))
        take_self = a_lt == want_self
        keys = jnp.where(take_self, keys, pk)
        idx = jnp.where(take_self, idx, pi)
        wrap = j == 0
        k2 = jnp.where(wrap, k + 1, k)
        j2 = jnp.where(wrap, k, j - 1)
        return k2, j2, keys, idx

    _, _, _, idx = lax.fori_loop(
        0, NROUNDS, round_body,
        (jnp.int32(1), jnp.int32(0), keys0, idx0))
    idx_ref[:] = idx


def _sort(keys):
    return pl.pallas_call(
        _sort_body,
        out_shape=jax.ShapeDtypeStruct((B, NP), jnp.int32),
    )(keys)


def _sc_body(table_hbm, idx_hbm, out_hbm, idx_row, buf0, buf1, out_v, sem0, sem1):
    wid = lax.axis_index("s") * 2 + lax.axis_index("c")
    bufs = (buf0, buf1)
    sems = (sem0, sem1)

    def start(slot, ch):
        pltpu.async_copy(
            table_hbm.at[idx_row.at[pl.ds(ch * CHUNK, CHUNK)]],
            bufs[slot], sems[slot])

    def wait(slot, ch):
        pltpu.make_async_copy(
            table_hbm.at[idx_row.at[pl.ds(ch * CHUNK, CHUNK)]],
            bufs[slot], sems[slot]).wait()

    def scan_chunk(buf, carry):
        def step(r, cy):
            (Trl, Trh, Til, Tih,
             Arl, Arh, Ail, Aih,
             Rrl, Rrh, Ril, Rih) = cy
            arl = buf[r, pl.ds(0, 16)]
            arh = buf[r, pl.ds(16, 16)]
            ail = buf[r, pl.ds(32, 16)]
            aih = buf[r, pl.ds(48, 16)]
            rrl = buf[r, pl.ds(64, 16)]
            rrh = buf[r, pl.ds(80, 16)]
            srl = arl * Trl - ail * Til
            sil = arl * Til + ail * Trl
            srh = arh * Trh - aih * Tih
            sih = arh * Tih + aih * Trh
            Arl = Arl + srl
            Ail = Ail + sil
            Arh = Arh + srh
            Aih = Aih + sih
            Rrl = Rrl + srl * rrl
            Ril = Ril + sil * rrl
            Rrh = Rrh + srh * rrh
            Rih = Rih + sih * rrh
            nTrl = Trl - srl
            nTil = Til - sil
            nTrh = Trh - srh
            nTih = Tih - sih
            return (nTrl, nTrh, nTil, nTih,
                    Arl, Arh, Ail, Aih,
                    Rrl, Rrh, Ril, Rih)

        return lax.fori_loop(0, CHUNK, step, carry)

    for rowi in range(ROWS_PER_WORKER):
        b = wid * ROWS_PER_WORKER + rowi
        pltpu.sync_copy(idx_hbm.at[b], idx_row)
        start(0, 0)

        ones = jnp.ones((16,), jnp.float32)
        zeros = jnp.zeros((16,), jnp.float32)
        carry = (ones, ones, zeros, zeros,
                 zeros, zeros, zeros, zeros,
                 zeros, zeros, zeros, zeros)

        def outer(ch2, cy):
            ch0 = ch2 * 2
            wait(0, ch0)
            start(1, ch0 + 1)
            cy = scan_chunk(buf0, cy)
            wait(1, ch0 + 1)

            @pl.when(ch2 < NCHUNK // 2 - 1)
            def _():
                start(0, ch0 + 2)

            return scan_chunk(buf1, cy)

        carry = lax.fori_loop(0, NCHUNK // 2, outer, carry)
        (_, _, _, _, Arl, Arh, Ail, Aih, Rrl, Rrh, Ril, Rih) = carry

        rel = Rrl + 1.0 - Arl
        reh = Rrh + 1.0 - Arh
        iml = Ril - Ail
        imh = Rih - Aih
        out_v[pl.ds(0, 16)] = rel * rel + iml * iml
        out_v[pl.ds(16, 16)] = reh * reh + imh * imh
        pltpu.sync_copy(out_v, out_hbm.at[b, :])


def _sc_call(table, sidx):
    fn = pl.kernel(
        _sc_body,
        out_type=jax.ShapeDtypeStruct((B, C), jnp.float32),
        mesh=plsc.VectorSubcoreMesh(
            core_axis_name="c", subcore_axis_name="s",
            num_cores=2, num_subcores=16,
        ),
        scratch_types=[
            pltpu.VMEM((NP,), jnp.int32),
            pltpu.VMEM((CHUNK, D), jnp.float32),
            pltpu.VMEM((CHUNK, D), jnp.float32),
            pltpu.VMEM((C,), jnp.float32),
            pltpu.SemaphoreType.DMA,
            pltpu.SemaphoreType.DMA,
        ],
    )
    return fn(table, sidx)


def kernel(position_grid, means2D, cov2d, rsrp, opacity, phi_o, depths, embedded):
    del position_grid
    padn = NP - N
    mx = jnp.pad(means2D[:, 0], (0, padn))
    my = jnp.pad(means2D[:, 1], (0, padn))
    c00 = jnp.pad(cov2d[:, 0, 0], (0, padn))
    c11 = jnp.pad(cov2d[:, 1, 1], (0, padn))
    cs = jnp.pad(cov2d[:, 0, 1] + cov2d[:, 1, 0], (0, padn))
    opa = jnp.pad(opacity[:, 0], (0, padn))
    zcol = jnp.zeros((NP,), jnp.float32)
    geo = jnp.stack([mx, my, c00, c11, cs, opa, zcol, zcol], axis=1)
    phi_p = jnp.pad(phi_o, ((0, padn), (0, 0)))
    rsrp_p = jnp.pad(rsrp, ((0, padn), (0, 0)))
    emb_p = jnp.pad(embedded, ((0, padn), (0, 0)))

    table = _prep(geo, phi_p, rsrp_p, emb_p)
    keys = jnp.pad(depths[..., 0], ((0, 0), (0, padn)),
                   constant_values=1e30)
    sidx = _sort(keys)
    return _sc_call(table, sidx)
